# Initial kernel scaffold; baseline (speedup 1.0000x reference)
import functools

import jax
import jax.numpy as jnp
from jax import lax
from jax.experimental import pallas as pl
from jax.experimental.pallas import tpu as pltpu
from jax.experimental.pallas import tpu_sc as plsc

_NUM_CORES = 2
_NUM_SUBCORES = 16
_NUM_TILES = _NUM_CORES * _NUM_SUBCORES
_CHUNK = 128


def _scatter_partials(n_acc, d, chunks_per_tile, rows_per_tile):
    mesh = plsc.VectorSubcoreMesh(
        core_axis_name="c", subcore_axis_name="s", num_cores=_NUM_CORES
    )

    @functools.partial(
        pl.kernel,
        out_type=jax.ShapeDtypeStruct((_NUM_CORES, n_acc, d), jnp.float32),
        mesh=mesh,
        scratch_types=[
            pltpu.VMEM_SHARED((n_acc, d), jnp.float32),
            pltpu.VMEM((_CHUNK,), jnp.int32),
            pltpu.VMEM((_CHUNK,), jnp.int32),
            pltpu.VMEM((_CHUNK, d), jnp.float32),
            pltpu.SemaphoreType.DMA,
        ],
    )
    def sc_kernel(feat_hbm, src_hbm, dst_hbm, zeros_hbm, out_hbm,
                  acc, src_v, dst_v, rows_v, sem):
        c = lax.axis_index("c")
        s = lax.axis_index("s")

        @pl.when(s == 0)
        def _():
            pltpu.sync_copy(zeros_hbm, acc)

        plsc.subcore_barrier()

        tile_base = (c * _NUM_SUBCORES + s) * chunks_per_tile * _CHUNK

        def chunk_body(i, carry):
            base = tile_base + i * _CHUNK
            pltpu.sync_copy(src_hbm.at[pl.ds(base, _CHUNK)], src_v)
            pltpu.sync_copy(dst_hbm.at[pl.ds(base, _CHUNK)], dst_v)
            pltpu.async_copy(feat_hbm.at[src_v], rows_v, sem).wait()
            pltpu.sync_copy(rows_v, acc.at[dst_v], add=True)
            return carry

        lax.fori_loop(0, chunks_per_tile, chunk_body, 0)

        plsc.subcore_barrier()

        row0 = s * rows_per_tile
        pltpu.sync_copy(
            acc.at[pl.ds(row0, rows_per_tile)],
            out_hbm.at[c].at[pl.ds(row0, rows_per_tile)],
        )

    return sc_kernel


def _combine_linear(partials, w, b2d, n, d_out, block_rows):
    def body(p_ref, w_ref, b_ref, o_ref):
        h = p_ref[0] + p_ref[1]
        o_ref[...] = jnp.maximum(
            jnp.dot(h, w_ref[...], preferred_element_type=jnp.float32)
            + b_ref[...],
            0.0,
        )

    grid = n // block_rows
    return pl.pallas_call(
        body,
        out_shape=jax.ShapeDtypeStruct((n, d_out), jnp.float32),
        grid=(grid,),
        in_specs=[
            pl.BlockSpec(
                (_NUM_CORES, block_rows, partials.shape[2]),
                lambda i: (0, i, 0),
            ),
            pl.BlockSpec(w.shape, lambda i: (0, 0)),
            pl.BlockSpec(b2d.shape, lambda i: (0, 0)),
        ],
        out_specs=pl.BlockSpec((block_rows, d_out), lambda i: (i, 0)),
    )(partials, w, b2d)


def kernel(feature, edge_index, W, b):
    n, d = feature.shape
    e = edge_index.shape[1]
    d_out = W.shape[1]

    per_tile = _NUM_TILES * _CHUNK
    e_pad = ((e + per_tile - 1) // per_tile) * per_tile
    chunks_per_tile = e_pad // per_tile
    src = edge_index[0]
    dst = edge_index[1]
    if e_pad != e:
        pad = e_pad - e
        src = jnp.concatenate([src, jnp.zeros((pad,), jnp.int32)])
        dst = jnp.concatenate([dst, jnp.full((pad,), n, jnp.int32)])

    n_acc = ((n + 1 + _NUM_SUBCORES - 1) // _NUM_SUBCORES) * _NUM_SUBCORES
    rows_per_tile = n_acc // _NUM_SUBCORES
    zeros = jnp.zeros((n_acc, d), jnp.float32)

    partials = _scatter_partials(n_acc, d, chunks_per_tile, rows_per_tile)(
        feature, src, dst, zeros
    )
    partials = partials[:, :n, :]

    b2d = b.reshape(1, d_out)
    block_rows = 2000 if n % 2000 == 0 else n
    return _combine_linear(partials, W, b2d, n, d_out, block_rows)


# speedup vs baseline: 4.0246x; 4.0246x over previous
import functools

import jax
import jax.numpy as jnp
from jax import lax
from jax.experimental import pallas as pl
from jax.experimental.pallas import tpu as pltpu
from jax.experimental.pallas import tpu_sc as plsc

_NUM_CORES = 2
_NUM_SUBCORES = 16
_NUM_TILES = _NUM_CORES * _NUM_SUBCORES
_CHUNK = 128


def _scatter_partials(n_acc, d, chunks_per_tile, rows_per_tile):
    mesh = plsc.VectorSubcoreMesh(
        core_axis_name="c", subcore_axis_name="s", num_cores=_NUM_CORES
    )

    @functools.partial(
        pl.kernel,
        out_type=jax.ShapeDtypeStruct((_NUM_CORES, n_acc, d), jnp.float32),
        mesh=mesh,
        scratch_types=[
            pltpu.VMEM_SHARED((n_acc, d), jnp.float32),
            pltpu.VMEM((_CHUNK,), jnp.int32),
            pltpu.VMEM((_CHUNK,), jnp.int32),
            pltpu.VMEM((_CHUNK, d), jnp.float32),
            pltpu.SemaphoreType.DMA,
        ],
    )
    def sc_kernel(feat_hbm, src_hbm, dst_hbm, zeros_hbm, out_hbm,
                  acc, src_v, dst_v, rows_v, sem):
        c = lax.axis_index("c")
        s = lax.axis_index("s")

        @pl.when(s == 0)
        def _():
            pltpu.sync_copy(zeros_hbm, acc)

        plsc.subcore_barrier()

        tile_base = (c * _NUM_SUBCORES + s) * chunks_per_tile * _CHUNK

        def chunk_body(i, carry):
            base = tile_base + i * _CHUNK
            pltpu.sync_copy(src_hbm.at[pl.ds(base, _CHUNK)], src_v)
            pltpu.sync_copy(dst_hbm.at[pl.ds(base, _CHUNK)], dst_v)
            pltpu.async_copy(feat_hbm.at[src_v], rows_v, sem).wait()
            pltpu.sync_copy(rows_v, acc.at[dst_v], add=True)
            return carry

        lax.fori_loop(0, chunks_per_tile, chunk_body, 0)

        plsc.subcore_barrier()

        row0 = s * rows_per_tile
        pltpu.sync_copy(
            acc.at[pl.ds(row0, rows_per_tile)],
            out_hbm.at[c].at[pl.ds(row0, rows_per_tile)],
        )

    return sc_kernel


def _combine_linear(partials, w, b2d, n, d_out, block_rows):
    def body(p_ref, w_ref, b_ref, o_ref):
        h = p_ref[0] + p_ref[1]
        o_ref[...] = jnp.maximum(
            jnp.dot(h, w_ref[...], preferred_element_type=jnp.float32)
            + b_ref[...],
            0.0,
        )

    grid = n // block_rows
    return pl.pallas_call(
        body,
        out_shape=jax.ShapeDtypeStruct((n, d_out), jnp.float32),
        grid=(grid,),
        in_specs=[
            pl.BlockSpec(
                (_NUM_CORES, block_rows, partials.shape[2]),
                lambda i: (0, i, 0),
            ),
            pl.BlockSpec(w.shape, lambda i: (0, 0)),
            pl.BlockSpec(b2d.shape, lambda i: (0, 0)),
        ],
        out_specs=pl.BlockSpec((block_rows, d_out), lambda i: (i, 0)),
    )(partials, w, b2d)


def kernel(feature, edge_index, W, b):
    n, d = feature.shape
    e = edge_index.shape[1]
    d_out = W.shape[1]

    per_tile = _NUM_TILES * _CHUNK
    e_pad = ((e + per_tile - 1) // per_tile) * per_tile
    chunks_per_tile = e_pad // per_tile
    src = edge_index[0]
    dst = edge_index[1]
    if e_pad != e:
        pad = e_pad - e
        src = jnp.concatenate([src, jnp.zeros((pad,), jnp.int32)])
        dst = jnp.concatenate([dst, jnp.full((pad,), n, jnp.int32)])

    quantum = _NUM_SUBCORES * 8
    n_acc = ((n + 1 + quantum - 1) // quantum) * quantum
    rows_per_tile = n_acc // _NUM_SUBCORES
    zeros = jnp.zeros((n_acc, d), jnp.float32)

    partials = _scatter_partials(n_acc, d, chunks_per_tile, rows_per_tile)(
        feature, src, dst, zeros
    )
    partials = partials[:, :n, :]

    b2d = b.reshape(1, d_out)
    block_rows = 2000 if n % 2000 == 0 else n
    return _combine_linear(partials, W, b2d, n, d_out, block_rows)
